# Initial kernel scaffold; baseline (speedup 1.0000x reference)
import jax
import jax.numpy as jnp
from jax.experimental import pallas as pl


def kernel(user_idx, item_idx, neg_item_idx, up_edges_0, up_edges_1, ip_edges_0, ip_edges_1, feat_user, feat_item, W_u0, W_u1, W_i0, W_i1, attU_W1, attU_b1, attU_W2, attI_W1, attI_b1, attI_W2, ln_g, ln_b):
    raise NotImplementedError("write your pallas kernel here")


# speedup vs baseline: 5.5514x; 5.5514x over previous
import functools
import jax
import jax.numpy as jnp
from jax import lax
from jax.experimental import pallas as pl
from jax.experimental.pallas import tpu as pltpu
from jax.experimental.pallas import tpu_sc as plsc

N = 10000
E = 320000
D = 128
ER = E // 128
NC = 2
NS = 16
RPT = N // NS

f32 = jnp.float32
i32 = jnp.int32


def _deg_body(edges, out, shared, idxb, onesb, zbuf):
    c = lax.axis_index("c")
    s = lax.axis_index("s")
    stream = s % 4
    role = s // 4
    graph = c * 2 + stream // 2
    end = stream % 2

    def fill(i, _):
        onesb[pl.ds(i * 16, 16)] = jnp.full((16,), 1.0, f32)
        return 0
    lax.fori_loop(0, 8, fill, 0)

    def zfill(i, _):
        zbuf[pl.ds(i * 16, 16)] = jnp.zeros((16,), f32)
        return 0
    lax.fori_loop(0, 2000 // 16, zfill, 0)

    @pl.when(role == 0)
    def _():
        for j in range(5):
            pltpu.sync_copy(zbuf, shared.at[pl.ds(stream * N + j * 2000, 2000)])

    plsc.subcore_barrier()

    rows_per_role = ER // 4
    base = role * rows_per_role
    seg = stream * N

    def chunk(i, _):
        pltpu.sync_copy(edges.at[graph, end, base + i], idxb)
        for l in range(8):
            v = idxb[pl.ds(l * 16, 16)]
            idxb[pl.ds(l * 16, 16)] = v + seg
        pltpu.sync_copy(onesb, shared.at[idxb], add=True)
        return 0
    lax.fori_loop(0, rows_per_role, chunk, 0)

    plsc.subcore_barrier()

    @pl.when(role == 0)
    def _():
        for j in range(5):
            pltpu.sync_copy(shared.at[pl.ds(stream * N + j * 2000, 2000)],
                            zbuf)
            pltpu.sync_copy(
                zbuf, out.at[pl.ds((graph * 2 + end) * N + j * 2000, 2000)])


def _deg_call(edges):
    mesh = plsc.VectorSubcoreMesh(core_axis_name="c", subcore_axis_name="s")
    return pl.kernel(
        _deg_body,
        out_type=jax.ShapeDtypeStruct((8 * N,), f32),
        mesh=mesh,
        scratch_types=[
            pltpu.VMEM_SHARED((4 * N,), f32),
            pltpu.VMEM((128,), i32),
            pltpu.VMEM((128,), f32),
            pltpu.VMEM((2000,), f32),
        ],
    )(edges)


def _msg_body(pair, h2, edges, out, agg_sh, srcb, dstb, rows, sem):
    c = lax.axis_index("c")
    s = lax.axis_index("s")
    graph = pair * 2 + c

    def zfill(i, _):
        rows[i // 8, pl.ds((i % 8) * 16, 16)] = jnp.zeros((16,), f32)
        return 0
    lax.fori_loop(0, 128 * 8, zfill, 0)

    start = s * 624

    @pl.when(s < NS - 1)
    def _():
        for j in range(4):
            pltpu.sync_copy(rows, agg_sh.at[pl.ds(start + j * 128, 128)])
        pltpu.sync_copy(rows.at[pl.ds(0, 112)],
                        agg_sh.at[pl.ds(start + 512, 112)])

    @pl.when(s == NS - 1)
    def _():
        for j in range(5):
            pltpu.sync_copy(rows, agg_sh.at[pl.ds(9360 + j * 128, 128)])

    plsc.subcore_barrier()

    my_n = (ER - s + NS - 1) // NS

    off = c * N

    def chunk(k, _):
        row0 = s + k * NS
        pltpu.sync_copy(edges.at[graph, 0, row0], srcb)
        pltpu.sync_copy(edges.at[graph, 1, row0], dstb)
        for l in range(8):
            v = srcb[pl.ds(l * 16, 16)]
            srcb[pl.ds(l * 16, 16)] = v + off
        pltpu.async_copy(h2.at[srcb], rows, sem).wait()
        pltpu.sync_copy(rows, agg_sh.at[dstb], add=True)
        return 0
    lax.fori_loop(0, my_n, chunk, 0)

    plsc.subcore_barrier()

    @pl.when(s < NS - 1)
    def _():
        pltpu.sync_copy(agg_sh.at[pl.ds(start, 624)],
                        out.at[c, pl.ds(start, 624)])

    @pl.when(s == NS - 1)
    def _():
        pltpu.sync_copy(agg_sh.at[pl.ds(9360, 640)],
                        out.at[c, pl.ds(9360, 640)])


def _msg_call(pair, h2, edges):
    mesh = plsc.VectorSubcoreMesh(core_axis_name="c", subcore_axis_name="s")
    return pl.kernel(
        functools.partial(_msg_body, pair),
        out_type=jax.ShapeDtypeStruct((2, N, D), f32),
        mesh=mesh,
        scratch_types=[
            pltpu.VMEM_SHARED((N, D), f32),
            pltpu.VMEM((128,), i32),
            pltpu.VMEM((128,), i32),
            pltpu.VMEM((128, D), f32),
            pltpu.SemaphoreType.DMA,
        ],
    )(h2, edges)


def _h_body(col0, col1, x, deg_t, Wa, Wb, out):
    xv = x[...]
    def ns(col):
        d = deg_t[:, col:col + 1]
        return jnp.where(d > 0.0, lax.rsqrt(jnp.maximum(d, 1.0)), 0.0)
    out[0:N, :] = jnp.dot(xv, Wa[...], preferred_element_type=f32) * ns(col0)
    out[N:2 * N, :] = jnp.dot(xv, Wb[...], preferred_element_type=f32) * ns(col1)


def _h_call(col0, col1, x, deg_t, Wa, Wb):
    return pl.pallas_call(
        functools.partial(_h_body, col0, col1),
        out_shape=jax.ShapeDtypeStruct((2 * N, D), f32),
    )(x, deg_t, Wa, Wb)


def _post_body(c0, c1, agg, deg_t, W1, b1, W2, g, b, out):
    def nd(col):
        d = deg_t[:, col:col + 1]
        return jnp.where(d > 0.0, lax.rsqrt(jnp.maximum(d, 1.0)), 0.0)

    def metapath(i, col):
        m = agg[i] * nd(col)
        return jnp.where(m >= 0.0, m, 0.01 * m)

    m0 = metapath(0, c0)
    m1 = metapath(1, c1)

    W1v = W1[...]
    b1v = b1[...]
    W2v = W2[...]

    def score(m):
        t = jnp.tanh(jnp.dot(m, W1v, preferred_element_type=f32) + b1v[None, :])
        w = jnp.dot(t, W2v, preferred_element_type=f32)
        return jnp.mean(w)

    w0 = score(m0)
    w1 = score(m1)
    mx = jnp.maximum(w0, w1)
    e0 = jnp.exp(w0 - mx)
    e1 = jnp.exp(w1 - mx)
    bsum = e0 + e1
    emb = (e0 / bsum) * m0 + (e1 / bsum) * m1
    emb = jnp.maximum(emb, 0.0)
    mu = jnp.mean(emb, axis=-1, keepdims=True)
    var = jnp.mean((emb - mu) ** 2, axis=-1, keepdims=True)
    out[...] = (emb - mu) * lax.rsqrt(var + 1e-5) * g[...][None, :] + b[...][None, :]


def _post_call(c0, c1, agg, deg_t, W1, b1, W2, g, b):
    return pl.pallas_call(
        functools.partial(_post_body, c0, c1),
        out_shape=jax.ShapeDtypeStruct((N, D), f32),
    )(agg, deg_t, W1, b1, W2, g, b)


_BW = 1024 // (NC * NS)


def _gather_body(uemb, iemb, uidx, iidx, nidx, ou, oi, on, idxv, rows, sem):
    c = lax.axis_index("c")
    s = lax.axis_index("s")
    wid = s * NC + c
    base = wid * _BW
    for table, idx_hbm, out in ((uemb, uidx, ou), (iemb, iidx, oi),
                                (iemb, nidx, on)):
        pltpu.sync_copy(idx_hbm.at[pl.ds(base, _BW)], idxv)
        pltpu.async_copy(table.at[idxv], rows, sem).wait()
        pltpu.sync_copy(rows, out.at[pl.ds(base, _BW)])


def _gather_call(uemb, iemb, uidx, iidx, nidx):
    mesh = plsc.VectorSubcoreMesh(core_axis_name="c", subcore_axis_name="s")
    sds = jax.ShapeDtypeStruct((1024, D), f32)
    return pl.kernel(
        _gather_body,
        out_type=(sds, sds, sds),
        mesh=mesh,
        scratch_types=[
            pltpu.VMEM((_BW,), i32),
            pltpu.VMEM((_BW, D), f32),
            pltpu.SemaphoreType.DMA,
        ],
    )(uemb, iemb, uidx, iidx, nidx)


def kernel(user_idx, item_idx, neg_item_idx, up_edges_0, up_edges_1,
           ip_edges_0, ip_edges_1, feat_user, feat_item, W_u0, W_u1, W_i0,
           W_i1, attU_W1, attU_b1, attU_W2, attI_W1, attI_b1, attI_W2,
           ln_g, ln_b):
    edges = jnp.stack([up_edges_0, up_edges_1, ip_edges_0, ip_edges_1])
    edges = edges.astype(i32).reshape(4, 2, ER, 128)

    deg = _deg_call(edges).reshape(8, N)
    deg_t = jnp.transpose(deg)

    h_u = _h_call(0, 2, feat_user, deg_t, W_u0, W_u1)
    h_i = _h_call(4, 6, feat_item, deg_t, W_i0, W_i1)

    agg_u = _msg_call(0, h_u, edges)
    agg_i = _msg_call(1, h_i, edges)

    user_emb = _post_call(1, 3, agg_u, deg_t, attU_W1, attU_b1, attU_W2,
                          ln_g, ln_b)
    item_emb = _post_call(5, 7, agg_i, deg_t, attI_W1, attI_b1, attI_W2,
                          ln_g, ln_b)

    return _gather_call(user_emb, item_emb,
                        user_idx.astype(i32), item_idx.astype(i32),
                        neg_item_idx.astype(i32))


# speedup vs baseline: 10.6567x; 1.9196x over previous
import functools
import jax
import jax.numpy as jnp
from jax import lax
from jax.experimental import pallas as pl
from jax.experimental.pallas import tpu as pltpu
from jax.experimental.pallas import tpu_sc as plsc

N = 10000
E = 320000
D = 128
ER = E // 128
NC = 2
NS = 16
RPT = N // NS

f32 = jnp.float32
i32 = jnp.int32


def _deg_body(edges, out, shared, idxbig, onesb, zbuf, ssem):
    c = lax.axis_index("c")
    s = lax.axis_index("s")
    stream = s % 4
    role = s // 4
    graph = c * 2 + stream // 2
    end = stream % 2

    def fill(i, _):
        onesb[pl.ds(i * 16, 16)] = jnp.full((16,), 1.0, f32)
        return 0
    lax.fori_loop(0, 8, fill, 0)

    def zfill(i, _):
        zbuf[pl.ds(i * 16, 16)] = jnp.zeros((16,), f32)
        return 0
    lax.fori_loop(0, 2000 // 16, zfill, 0)

    @pl.when(role == 0)
    def _():
        for j in range(5):
            pltpu.sync_copy(zbuf, shared.at[pl.ds(stream * N + j * 2000, 2000)])

    plsc.subcore_barrier()

    base = role * 624
    seg = stream * N

    def adjust(p, j):
        for l in range(8):
            v = idxbig[p, j, pl.ds(l * 16, 16)]
            idxbig[p, j, pl.ds(l * 16, 16)] = v + seg

    def swait(j):
        pltpu.make_async_copy(out.at[pl.ds(0, 128)],
                              zbuf.at[pl.ds(0, 128)], ssem.at[j]).wait()

    def half(u, p):
        row0 = pl.multiple_of(base + u * 16 + 8 * p, 8)
        pltpu.sync_copy(edges.at[graph, end, pl.ds(row0, 8)], idxbig.at[p])
        for j in range(8):
            adjust(p, j)
        for j in range(8):
            if p == 0 and j < 4:
                @pl.when(u >= 1)
                def _(j=j):
                    swait(j % 4)
            else:
                swait(j % 4)
            pltpu.async_copy(onesb, shared.at[idxbig.at[p, j]],
                             ssem.at[j % 4], add=True)

    def body(u, _):
        half(u, 0)
        half(u, 1)
        return 0
    lax.fori_loop(0, 39, body, 0)

    @pl.when(role == 3)
    def _():
        for t in range(4):
            pltpu.sync_copy(edges.at[graph, end, 2496 + t], idxbig.at[0, t])
            adjust(0, t)
            swait(t)
            pltpu.async_copy(onesb, shared.at[idxbig.at[0, t]],
                             ssem.at[t], add=True)

    for j in range(4):
        swait(j)

    plsc.subcore_barrier()

    @pl.when(role == 0)
    def _():
        for j in range(5):
            pltpu.sync_copy(shared.at[pl.ds(stream * N + j * 2000, 2000)],
                            zbuf)
            pltpu.sync_copy(
                zbuf, out.at[pl.ds((graph * 2 + end) * N + j * 2000, 2000)])


def _deg_call(edges):
    mesh = plsc.VectorSubcoreMesh(core_axis_name="c", subcore_axis_name="s")
    return pl.kernel(
        _deg_body,
        out_type=jax.ShapeDtypeStruct((8 * N,), f32),
        mesh=mesh,
        scratch_types=[
            pltpu.VMEM_SHARED((4 * N,), f32),
            pltpu.VMEM((2, 8, 128), i32),
            pltpu.VMEM((128,), f32),
            pltpu.VMEM((2000,), f32),
            pltpu.SemaphoreType.DMA((4,)),
        ],
    )(edges)


def _msg_body(pair, h2, edges, out, agg_sh, srcbig, dstbig, rows,
              gsem, ssem):
    c = lax.axis_index("c")
    s = lax.axis_index("s")

    def zfill(i, _):
        rows[0, i // 8, pl.ds((i % 8) * 16, 16)] = jnp.zeros((16,), f32)
        return 0
    lax.fori_loop(0, 128 * 8, zfill, 0)

    start = s * 624
    zslab = rows.at[0]

    @pl.when(s < NS - 1)
    def _():
        for j in range(4):
            pltpu.sync_copy(zslab, agg_sh.at[pl.ds(start + j * 128, 128)])
        pltpu.sync_copy(rows.at[0, pl.ds(0, 112)],
                        agg_sh.at[pl.ds(start + 512, 112)])

    @pl.when(s == NS - 1)
    def _():
        for j in range(5):
            pltpu.sync_copy(zslab, agg_sh.at[pl.ds(9360 + j * 128, 128)])

    plsc.subcore_barrier()

    rstart = jnp.where(s < 8, s * 160, 1280 + (s - 8) * 152)
    m = jnp.where(s < 8, 160, 152)
    g = pair * 2 + c
    off = c * N

    def swait(b):
        pltpu.make_async_copy(h2.at[pl.ds(0, 128)], rows.at[b],
                              ssem.at[b]).wait()

    def adjust(pb, j):
        for l in range(8):
            v = srcbig[pb, j, pl.ds(l * 16, 16)]
            srcbig[pb, j, pl.ds(l * 16, 16)] = v + off

    def group(u, q):
        pb = q // 4
        r0 = (q % 4) * 2
        baseq = u * 16 + 2 * q

        @pl.when(baseq < m)
        def _():
            if q % 4 == 0:
                row0 = pl.multiple_of(rstart + u * 16 + 8 * pb, 8)
                pltpu.sync_copy(edges.at[g, 0, pl.ds(row0, 8)],
                                srcbig.at[pb])
                pltpu.sync_copy(edges.at[g, 1, pl.ds(row0, 8)],
                                dstbig.at[pb])
                for j in range(8):
                    adjust(pb, j)
            descs = [None] * 2
            for b in range(2):
                k = baseq + b

                @pl.when(k < m)
                def _(b=b, k=k):
                    @pl.when(k >= 2)
                    def _():
                        swait(b)
                    descs[b] = pltpu.async_copy(
                        h2.at[srcbig.at[pb, r0 + b]],
                        rows.at[b], gsem.at[b])
            for b in range(2):
                k = baseq + b

                @pl.when(k < m)
                def _(b=b):
                    descs[b].wait()
                    pltpu.async_copy(
                        rows.at[b],
                        agg_sh.at[dstbig.at[pb, r0 + b]],
                        ssem.at[b], add=True)

    ub = (m + 15) // 16

    def body(u, _):
        for q in range(8):
            group(u, q)
        return 0
    lax.fori_loop(0, ub, body, 0)

    @pl.when(s == NS - 1)
    def _():
        for t in range(4):
            b = t % 2
            pltpu.sync_copy(edges.at[g, 0, 2496 + t], srcbig.at[0, t])
            pltpu.sync_copy(edges.at[g, 1, 2496 + t], dstbig.at[0, t])
            adjust(0, t)
            swait(b)
            d = pltpu.async_copy(h2.at[srcbig.at[0, t]], rows.at[b],
                                 gsem.at[b])
            d.wait()
            pltpu.async_copy(rows.at[b], agg_sh.at[dstbig.at[0, t]],
                             ssem.at[b], add=True)

    for b in range(2):
        swait(b)

    plsc.subcore_barrier()

    @pl.when(s < NS - 1)
    def _():
        pltpu.sync_copy(agg_sh.at[pl.ds(start, 624)],
                        out.at[c, pl.ds(start, 624)])

    @pl.when(s == NS - 1)
    def _():
        pltpu.sync_copy(agg_sh.at[pl.ds(9360, 640)],
                        out.at[c, pl.ds(9360, 640)])


def _msg_call(pair, h2, edges):
    mesh = plsc.VectorSubcoreMesh(core_axis_name="c", subcore_axis_name="s")
    return pl.kernel(
        functools.partial(_msg_body, pair),
        out_type=jax.ShapeDtypeStruct((2, N, D), f32),
        mesh=mesh,
        scratch_types=[
            pltpu.VMEM_SHARED((N, D), f32),
            pltpu.VMEM((2, 8, 128), i32),
            pltpu.VMEM((2, 8, 128), i32),
            pltpu.VMEM((2, 128, D), f32),
            pltpu.SemaphoreType.DMA((2,)),
            pltpu.SemaphoreType.DMA((2,)),
        ],
    )(h2, edges)


def _h_body(col0, col1, x, deg_t, Wa, Wb, out):
    xv = x[...]
    def ns(col):
        d = deg_t[:, col:col + 1]
        return jnp.where(d > 0.0, lax.rsqrt(jnp.maximum(d, 1.0)), 0.0)
    out[0:N, :] = jnp.dot(xv, Wa[...], preferred_element_type=f32) * ns(col0)
    out[N:2 * N, :] = jnp.dot(xv, Wb[...], preferred_element_type=f32) * ns(col1)


def _h_call(col0, col1, x, deg_t, Wa, Wb):
    return pl.pallas_call(
        functools.partial(_h_body, col0, col1),
        out_shape=jax.ShapeDtypeStruct((2 * N, D), f32),
    )(x, deg_t, Wa, Wb)


def _post_body(c0, c1, agg, deg_t, W1, b1, W2, g, b, out):
    def nd(col):
        d = deg_t[:, col:col + 1]
        return jnp.where(d > 0.0, lax.rsqrt(jnp.maximum(d, 1.0)), 0.0)

    def metapath(i, col):
        m = agg[i] * nd(col)
        return jnp.where(m >= 0.0, m, 0.01 * m)

    m0 = metapath(0, c0)
    m1 = metapath(1, c1)

    W1v = W1[...]
    b1v = b1[...]
    W2v = W2[...]

    def score(m):
        t = jnp.tanh(jnp.dot(m, W1v, preferred_element_type=f32) + b1v[None, :])
        w = jnp.dot(t, W2v, preferred_element_type=f32)
        return jnp.mean(w)

    w0 = score(m0)
    w1 = score(m1)
    mx = jnp.maximum(w0, w1)
    e0 = jnp.exp(w0 - mx)
    e1 = jnp.exp(w1 - mx)
    bsum = e0 + e1
    emb = (e0 / bsum) * m0 + (e1 / bsum) * m1
    emb = jnp.maximum(emb, 0.0)
    mu = jnp.mean(emb, axis=-1, keepdims=True)
    var = jnp.mean((emb - mu) ** 2, axis=-1, keepdims=True)
    out[...] = (emb - mu) * lax.rsqrt(var + 1e-5) * g[...][None, :] + b[...][None, :]


def _post_call(c0, c1, agg, deg_t, W1, b1, W2, g, b):
    return pl.pallas_call(
        functools.partial(_post_body, c0, c1),
        out_shape=jax.ShapeDtypeStruct((N, D), f32),
    )(agg, deg_t, W1, b1, W2, g, b)


_BW = 1024 // (NC * NS)


def _gather_body(uemb, iemb, uidx, iidx, nidx, ou, oi, on, idxv, rows, sem):
    c = lax.axis_index("c")
    s = lax.axis_index("s")
    wid = s * NC + c
    base = wid * _BW
    for table, idx_hbm, out in ((uemb, uidx, ou), (iemb, iidx, oi),
                                (iemb, nidx, on)):
        pltpu.sync_copy(idx_hbm.at[pl.ds(base, _BW)], idxv)
        pltpu.async_copy(table.at[idxv], rows, sem).wait()
        pltpu.sync_copy(rows, out.at[pl.ds(base, _BW)])


def _gather_call(uemb, iemb, uidx, iidx, nidx):
    mesh = plsc.VectorSubcoreMesh(core_axis_name="c", subcore_axis_name="s")
    sds = jax.ShapeDtypeStruct((1024, D), f32)
    return pl.kernel(
        _gather_body,
        out_type=(sds, sds, sds),
        mesh=mesh,
        scratch_types=[
            pltpu.VMEM((_BW,), i32),
            pltpu.VMEM((_BW, D), f32),
            pltpu.SemaphoreType.DMA,
        ],
    )(uemb, iemb, uidx, iidx, nidx)


def kernel(user_idx, item_idx, neg_item_idx, up_edges_0, up_edges_1,
           ip_edges_0, ip_edges_1, feat_user, feat_item, W_u0, W_u1, W_i0,
           W_i1, attU_W1, attU_b1, attU_W2, attI_W1, attI_b1, attI_W2,
           ln_g, ln_b):
    edges = jnp.stack([up_edges_0, up_edges_1, ip_edges_0, ip_edges_1])
    edges = edges.astype(i32).reshape(4, 2, ER, 128)

    deg = _deg_call(edges).reshape(8, N)
    deg_t = jnp.transpose(deg)

    h_u = _h_call(0, 2, feat_user, deg_t, W_u0, W_u1)
    h_i = _h_call(4, 6, feat_item, deg_t, W_i0, W_i1)

    agg_u = _msg_call(0, h_u, edges)
    agg_i = _msg_call(1, h_i, edges)

    user_emb = _post_call(1, 3, agg_u, deg_t, attU_W1, attU_b1, attU_W2,
                          ln_g, ln_b)
    item_emb = _post_call(5, 7, agg_i, deg_t, attI_W1, attI_b1, attI_W2,
                          ln_g, ln_b)

    return _gather_call(user_emb, item_emb,
                        user_idx.astype(i32), item_idx.astype(i32),
                        neg_item_idx.astype(i32))


# speedup vs baseline: 13.5143x; 1.2682x over previous
import functools
import jax
import jax.numpy as jnp
from jax import lax
from jax.experimental import pallas as pl
from jax.experimental.pallas import tpu as pltpu
from jax.experimental.pallas import tpu_sc as plsc

N = 10000
E = 320000
D = 128
ER = E // 128
NC = 2
NS = 16
RPT = N // NS

f32 = jnp.float32
i32 = jnp.int32


def _deg_body(edges, out, shared, idxbig, onesb, zbuf, ssem):
    c = lax.axis_index("c")
    s = lax.axis_index("s")
    stream = s % 4
    role = s // 4
    graph = c * 2 + stream // 2
    end = stream % 2

    def fill(i, _):
        onesb[pl.ds(i * 16, 16)] = jnp.full((16,), 1.0, f32)
        return 0
    lax.fori_loop(0, 8, fill, 0)

    def zfill(i, _):
        zbuf[pl.ds(i * 16, 16)] = jnp.zeros((16,), f32)
        return 0
    lax.fori_loop(0, 2000 // 16, zfill, 0)

    @pl.when(role == 0)
    def _():
        for j in range(5):
            pltpu.sync_copy(zbuf, shared.at[pl.ds(stream * N + j * 2000, 2000)])

    plsc.subcore_barrier()

    base = role * 624
    seg = stream * N

    def adjust(p, j):
        for l in range(8):
            v = idxbig[p, j, pl.ds(l * 16, 16)]
            idxbig[p, j, pl.ds(l * 16, 16)] = v + seg

    def swait(j):
        pltpu.make_async_copy(out.at[pl.ds(0, 128)],
                              zbuf.at[pl.ds(0, 128)], ssem.at[j]).wait()

    def half(u, p):
        row0 = pl.multiple_of(base + u * 16 + 8 * p, 8)
        pltpu.sync_copy(edges.at[graph, end, pl.ds(row0, 8)], idxbig.at[p])
        for j in range(8):
            adjust(p, j)
        for j in range(8):
            if p == 0 and j < 4:
                @pl.when(u >= 1)
                def _(j=j):
                    swait(j % 4)
            else:
                swait(j % 4)
            pltpu.async_copy(onesb, shared.at[idxbig.at[p, j]],
                             ssem.at[j % 4], add=True)

    def body(u, _):
        half(u, 0)
        half(u, 1)
        return 0
    lax.fori_loop(0, 39, body, 0)

    @pl.when(role == 3)
    def _():
        for t in range(4):
            pltpu.sync_copy(edges.at[graph, end, 2496 + t], idxbig.at[0, t])
            adjust(0, t)
            swait(t)
            pltpu.async_copy(onesb, shared.at[idxbig.at[0, t]],
                             ssem.at[t], add=True)

    for j in range(4):
        swait(j)

    plsc.subcore_barrier()

    @pl.when(role == 0)
    def _():
        for j in range(5):
            pltpu.sync_copy(shared.at[pl.ds(stream * N + j * 2000, 2000)],
                            zbuf)
            pltpu.sync_copy(
                zbuf, out.at[pl.ds((graph * 2 + end) * N + j * 2000, 2000)])


def _deg_call(edges):
    mesh = plsc.VectorSubcoreMesh(core_axis_name="c", subcore_axis_name="s")
    return pl.kernel(
        _deg_body,
        out_type=jax.ShapeDtypeStruct((8 * N,), f32),
        mesh=mesh,
        scratch_types=[
            pltpu.VMEM_SHARED((4 * N,), f32),
            pltpu.VMEM((2, 8, 128), i32),
            pltpu.VMEM((128,), f32),
            pltpu.VMEM((2000,), f32),
            pltpu.SemaphoreType.DMA((4,)),
        ],
    )(edges)


def _msg_body(pair, h2, edges, out, agg_sh, srcbig, dstbig, rows,
              gsem, ssem):
    c = lax.axis_index("c")
    s = lax.axis_index("s")

    def zfill(i, _):
        rows[0, i // 8, pl.ds((i % 8) * 16, 16)] = jnp.zeros((16,), f32)
        return 0
    lax.fori_loop(0, 128 * 8, zfill, 0)

    start = s * 624
    zslab = rows.at[0]

    @pl.when(s < NS - 1)
    def _():
        for j in range(4):
            pltpu.sync_copy(zslab, agg_sh.at[pl.ds(start + j * 128, 128)])
        pltpu.sync_copy(rows.at[0, pl.ds(0, 112)],
                        agg_sh.at[pl.ds(start + 512, 112)])

    @pl.when(s == NS - 1)
    def _():
        for j in range(5):
            pltpu.sync_copy(zslab, agg_sh.at[pl.ds(9360 + j * 128, 128)])

    plsc.subcore_barrier()

    rstart = jnp.where(s < 8, s * 160, 1280 + (s - 8) * 152)
    m = jnp.where(s < 8, 160, 152)
    g = pair * 2 + c
    off = c * N

    def swait(b):
        pltpu.make_async_copy(h2.at[pl.ds(0, 128)], rows.at[b],
                              ssem.at[b]).wait()

    def adjust(pb, j):
        for l in range(8):
            v = srcbig[pb, j, pl.ds(l * 16, 16)]
            srcbig[pb, j, pl.ds(l * 16, 16)] = v + off

    def gwait(b):
        pltpu.make_async_copy(h2.at[pl.ds(0, 128)], rows.at[b],
                              gsem.at[b]).wait()

    def step(u, q, b):
        k = u * 16 + 2 * q + b
        pb = q // 4
        islot = 2 * (q % 4) + b
        if b == 1:
            pbp, islotp = q // 4, 2 * (q % 4)
        elif q >= 1:
            pbp, islotp = (q - 1) // 4, 2 * ((q - 1) % 4) + 1
        else:
            pbp, islotp = 1, 7

        @pl.when(k < m)
        def _():
            if q % 4 == 0 and b == 0:
                row0 = pl.multiple_of(rstart + u * 16 + 8 * pb, 8)
                pltpu.sync_copy(edges.at[g, 0, pl.ds(row0, 8)],
                                srcbig.at[pb])
                pltpu.sync_copy(edges.at[g, 1, pl.ds(row0, 8)],
                                dstbig.at[pb])
                for j in range(8):
                    adjust(pb, j)

            @pl.when(k >= 2)
            def _():
                swait(b)
            pltpu.async_copy(h2.at[srcbig.at[pb, islot]], rows.at[b],
                             gsem.at[b])

            @pl.when(k >= 1)
            def _():
                gwait(1 - b)
                pltpu.async_copy(rows.at[1 - b],
                                 agg_sh.at[dstbig.at[pbp, islotp]],
                                 ssem.at[1 - b], add=True)

    ub = (m + 15) // 16

    def body(u, _):
        for q in range(8):
            for b in range(2):
                step(u, q, b)
        return 0
    lax.fori_loop(0, ub, body, 0)

    @pl.when(s < 8)
    def _():
        gwait(1)
        pltpu.async_copy(rows.at[1], agg_sh.at[dstbig.at[1, 7]],
                         ssem.at[1], add=True)

    @pl.when(s >= 8)
    def _():
        gwait(1)
        pltpu.async_copy(rows.at[1], agg_sh.at[dstbig.at[0, 7]],
                         ssem.at[1], add=True)

    @pl.when(s == NS - 1)
    def _():
        for t in range(4):
            b = t % 2
            pltpu.sync_copy(edges.at[g, 0, 2496 + t], srcbig.at[0, t])
            pltpu.sync_copy(edges.at[g, 1, 2496 + t], dstbig.at[0, t])
            adjust(0, t)
            swait(b)
            d = pltpu.async_copy(h2.at[srcbig.at[0, t]], rows.at[b],
                                 gsem.at[b])
            d.wait()
            pltpu.async_copy(rows.at[b], agg_sh.at[dstbig.at[0, t]],
                             ssem.at[b], add=True)

    for b in range(2):
        swait(b)

    plsc.subcore_barrier()

    @pl.when(s < NS - 1)
    def _():
        pltpu.sync_copy(agg_sh.at[pl.ds(start, 624)],
                        out.at[c, pl.ds(start, 624)])

    @pl.when(s == NS - 1)
    def _():
        pltpu.sync_copy(agg_sh.at[pl.ds(9360, 640)],
                        out.at[c, pl.ds(9360, 640)])


def _msg_call(pair, h2, edges):
    mesh = plsc.VectorSubcoreMesh(core_axis_name="c", subcore_axis_name="s")
    return pl.kernel(
        functools.partial(_msg_body, pair),
        out_type=jax.ShapeDtypeStruct((2, N, D), f32),
        mesh=mesh,
        scratch_types=[
            pltpu.VMEM_SHARED((N, D), f32),
            pltpu.VMEM((2, 8, 128), i32),
            pltpu.VMEM((2, 8, 128), i32),
            pltpu.VMEM((2, 128, D), f32),
            pltpu.SemaphoreType.DMA((2,)),
            pltpu.SemaphoreType.DMA((2,)),
        ],
    )(h2, edges)


def _h_body(col0, col1, x, deg_t, Wa, Wb, out):
    xv = x[...]
    def ns(col):
        d = deg_t[:, col:col + 1]
        return jnp.where(d > 0.0, lax.rsqrt(jnp.maximum(d, 1.0)), 0.0)
    out[0:N, :] = jnp.dot(xv, Wa[...], preferred_element_type=f32) * ns(col0)
    out[N:2 * N, :] = jnp.dot(xv, Wb[...], preferred_element_type=f32) * ns(col1)


def _h_call(col0, col1, x, deg_t, Wa, Wb):
    return pl.pallas_call(
        functools.partial(_h_body, col0, col1),
        out_shape=jax.ShapeDtypeStruct((2 * N, D), f32),
    )(x, deg_t, Wa, Wb)


def _post_body(c0, c1, agg, deg_t, W1, b1, W2, g, b, out):
    def nd(col):
        d = deg_t[:, col:col + 1]
        return jnp.where(d > 0.0, lax.rsqrt(jnp.maximum(d, 1.0)), 0.0)

    def metapath(i, col):
        m = agg[i] * nd(col)
        return jnp.where(m >= 0.0, m, 0.01 * m)

    m0 = metapath(0, c0)
    m1 = metapath(1, c1)

    W1v = W1[...]
    b1v = b1[...]
    W2v = W2[...]

    def score(m):
        t = jnp.tanh(jnp.dot(m, W1v, preferred_element_type=f32) + b1v[None, :])
        w = jnp.dot(t, W2v, preferred_element_type=f32)
        return jnp.mean(w)

    w0 = score(m0)
    w1 = score(m1)
    mx = jnp.maximum(w0, w1)
    e0 = jnp.exp(w0 - mx)
    e1 = jnp.exp(w1 - mx)
    bsum = e0 + e1
    emb = (e0 / bsum) * m0 + (e1 / bsum) * m1
    emb = jnp.maximum(emb, 0.0)
    mu = jnp.mean(emb, axis=-1, keepdims=True)
    var = jnp.mean((emb - mu) ** 2, axis=-1, keepdims=True)
    out[...] = (emb - mu) * lax.rsqrt(var + 1e-5) * g[...][None, :] + b[...][None, :]


def _post_call(c0, c1, agg, deg_t, W1, b1, W2, g, b):
    return pl.pallas_call(
        functools.partial(_post_body, c0, c1),
        out_shape=jax.ShapeDtypeStruct((N, D), f32),
    )(agg, deg_t, W1, b1, W2, g, b)


_BW = 1024 // (NC * NS)


def _gather_body(uemb, iemb, uidx, iidx, nidx, ou, oi, on, idxv, rows, sem):
    c = lax.axis_index("c")
    s = lax.axis_index("s")
    wid = s * NC + c
    base = wid * _BW
    for table, idx_hbm, out in ((uemb, uidx, ou), (iemb, iidx, oi),
                                (iemb, nidx, on)):
        pltpu.sync_copy(idx_hbm.at[pl.ds(base, _BW)], idxv)
        pltpu.async_copy(table.at[idxv], rows, sem).wait()
        pltpu.sync_copy(rows, out.at[pl.ds(base, _BW)])


def _gather_call(uemb, iemb, uidx, iidx, nidx):
    mesh = plsc.VectorSubcoreMesh(core_axis_name="c", subcore_axis_name="s")
    sds = jax.ShapeDtypeStruct((1024, D), f32)
    return pl.kernel(
        _gather_body,
        out_type=(sds, sds, sds),
        mesh=mesh,
        scratch_types=[
            pltpu.VMEM((_BW,), i32),
            pltpu.VMEM((_BW, D), f32),
            pltpu.SemaphoreType.DMA,
        ],
    )(uemb, iemb, uidx, iidx, nidx)


def kernel(user_idx, item_idx, neg_item_idx, up_edges_0, up_edges_1,
           ip_edges_0, ip_edges_1, feat_user, feat_item, W_u0, W_u1, W_i0,
           W_i1, attU_W1, attU_b1, attU_W2, attI_W1, attI_b1, attI_W2,
           ln_g, ln_b):
    edges = jnp.stack([up_edges_0, up_edges_1, ip_edges_0, ip_edges_1])
    edges = edges.astype(i32).reshape(4, 2, ER, 128)

    deg = _deg_call(edges).reshape(8, N)
    deg_t = jnp.transpose(deg)

    h_u = _h_call(0, 2, feat_user, deg_t, W_u0, W_u1)
    h_i = _h_call(4, 6, feat_item, deg_t, W_i0, W_i1)

    agg_u = _msg_call(0, h_u, edges)
    agg_i = _msg_call(1, h_i, edges)

    user_emb = _post_call(1, 3, agg_u, deg_t, attU_W1, attU_b1, attU_W2,
                          ln_g, ln_b)
    item_emb = _post_call(5, 7, agg_i, deg_t, attI_W1, attI_b1, attI_W2,
                          ln_g, ln_b)

    return _gather_call(user_emb, item_emb,
                        user_idx.astype(i32), item_idx.astype(i32),
                        neg_item_idx.astype(i32))


# speedup vs baseline: 14.1101x; 1.0441x over previous
import functools
import jax
import jax.numpy as jnp
from jax import lax
from jax.experimental import pallas as pl
from jax.experimental.pallas import tpu as pltpu
from jax.experimental.pallas import tpu_sc as plsc

N = 10000
E = 320000
D = 128
ER = E // 128
NC = 2
NS = 16
RPT = N // NS

f32 = jnp.float32
i32 = jnp.int32


def _deg_body(edges, out, shared, idxbig, onesb, zbuf, ssem):
    c = lax.axis_index("c")
    s = lax.axis_index("s")
    stream = s % 4
    role = s // 4
    graph = c * 2 + stream // 2
    end = stream % 2

    def fill(i, _):
        onesb[pl.ds(i * 16, 16)] = jnp.full((16,), 1.0, f32)
        return 0
    lax.fori_loop(0, 8, fill, 0)

    def zfill(i, _):
        zbuf[pl.ds(i * 16, 16)] = jnp.zeros((16,), f32)
        return 0
    lax.fori_loop(0, 2000 // 16, zfill, 0)

    @pl.when(role == 0)
    def _():
        for j in range(5):
            pltpu.sync_copy(zbuf, shared.at[pl.ds(stream * N + j * 2000, 2000)])

    plsc.subcore_barrier()

    base = role * 624
    seg = stream * N

    def adjust(p, j):
        for l in range(8):
            v = idxbig[p, j, pl.ds(l * 16, 16)]
            idxbig[p, j, pl.ds(l * 16, 16)] = v + seg

    def swait(j):
        pltpu.make_async_copy(out.at[pl.ds(0, 128)],
                              zbuf.at[pl.ds(0, 128)], ssem.at[j]).wait()

    def half(u, p):
        row0 = pl.multiple_of(base + u * 16 + 8 * p, 8)
        pltpu.sync_copy(edges.at[graph, end, pl.ds(row0, 8)], idxbig.at[p])
        for j in range(8):
            adjust(p, j)
        for j in range(8):
            if p == 0 and j < 4:
                @pl.when(u >= 1)
                def _(j=j):
                    swait(j % 4)
            else:
                swait(j % 4)
            pltpu.async_copy(onesb, shared.at[idxbig.at[p, j]],
                             ssem.at[j % 4], add=True)

    def body(u, _):
        half(u, 0)
        half(u, 1)
        return 0
    lax.fori_loop(0, 39, body, 0)

    @pl.when(role == 3)
    def _():
        for t in range(4):
            pltpu.sync_copy(edges.at[graph, end, 2496 + t], idxbig.at[0, t])
            adjust(0, t)
            swait(t)
            pltpu.async_copy(onesb, shared.at[idxbig.at[0, t]],
                             ssem.at[t], add=True)

    for j in range(4):
        swait(j)

    plsc.subcore_barrier()

    @pl.when(role == 0)
    def _():
        for j in range(5):
            pltpu.sync_copy(shared.at[pl.ds(stream * N + j * 2000, 2000)],
                            zbuf)
            pltpu.sync_copy(
                zbuf, out.at[pl.ds((graph * 2 + end) * N + j * 2000, 2000)])


def _deg_call(edges):
    mesh = plsc.VectorSubcoreMesh(core_axis_name="c", subcore_axis_name="s")
    return pl.kernel(
        _deg_body,
        out_type=jax.ShapeDtypeStruct((8 * N,), f32),
        mesh=mesh,
        scratch_types=[
            pltpu.VMEM_SHARED((4 * N,), f32),
            pltpu.VMEM((2, 8, 128), i32),
            pltpu.VMEM((128,), f32),
            pltpu.VMEM((2000,), f32),
            pltpu.SemaphoreType.DMA((4,)),
        ],
    )(edges)


def _msg_body(pair, h2, edges, out, agg_sh, srcbig, dstbig, rows,
              gsem, ssem, isem):
    c = lax.axis_index("c")
    s = lax.axis_index("s")

    def zfill(i, _):
        rows[0, i // 8, pl.ds((i % 8) * 16, 16)] = jnp.zeros((16,), f32)
        return 0
    lax.fori_loop(0, 128 * 8, zfill, 0)

    start = s * 624
    zslab = rows.at[0]

    @pl.when(s < NS - 1)
    def _():
        for j in range(4):
            pltpu.sync_copy(zslab, agg_sh.at[pl.ds(start + j * 128, 128)])
        pltpu.sync_copy(rows.at[0, pl.ds(0, 112)],
                        agg_sh.at[pl.ds(start + 512, 112)])

    @pl.when(s == NS - 1)
    def _():
        for j in range(5):
            pltpu.sync_copy(zslab, agg_sh.at[pl.ds(9360 + j * 128, 128)])

    plsc.subcore_barrier()

    rstart = jnp.where(s < 8, s * 160, 1280 + (s - 8) * 152)
    m = jnp.where(s < 8, 160, 152)
    g = pair * 2 + c
    off = c * N

    def swait(b):
        pltpu.make_async_copy(h2.at[pl.ds(0, 128)], rows.at[b],
                              ssem.at[b]).wait()

    def adjust(pb, j):
        for l in range(8):
            v = srcbig[pb, j, pl.ds(l * 16, 16)]
            srcbig[pb, j, pl.ds(l * 16, 16)] = v + off

    def gwait(b):
        pltpu.make_async_copy(h2.at[pl.ds(0, 128)], rows.at[b],
                              gsem.at[b]).wait()

    def step(u, q, b):
        k = u * 16 + 2 * q + b
        pb = q // 4
        islot = 2 * (q % 4) + b
        if b == 1:
            pbp, islotp = q // 4, 2 * (q % 4)
        elif q >= 1:
            pbp, islotp = (q - 1) // 4, 2 * ((q - 1) % 4) + 1
        else:
            pbp, islotp = 1, 7

        @pl.when(k < m)
        def _():
            if q % 4 == 0 and b == 0:
                pltpu.make_async_copy(edges.at[g, 0, pl.ds(0, 8)],
                                      srcbig.at[pb], isem).wait()
                pltpu.make_async_copy(edges.at[g, 1, pl.ds(0, 8)],
                                      dstbig.at[pb], isem).wait()
                for j in range(8):
                    adjust(pb, j)
            if b == 0 and q in (1, 5):
                tb = u * 16 + 8 if q == 1 else (u + 1) * 16
                pbt = 1 if q == 1 else 0

                @pl.when(tb < m)
                def _():
                    row0 = pl.multiple_of(rstart + tb, 8)
                    pltpu.async_copy(edges.at[g, 0, pl.ds(row0, 8)],
                                     srcbig.at[pbt], isem)
                    pltpu.async_copy(edges.at[g, 1, pl.ds(row0, 8)],
                                     dstbig.at[pbt], isem)

            @pl.when(k >= 2)
            def _():
                swait(b)
            pltpu.async_copy(h2.at[srcbig.at[pb, islot]], rows.at[b],
                             gsem.at[b])

            @pl.when(k >= 1)
            def _():
                gwait(1 - b)
                pltpu.async_copy(rows.at[1 - b],
                                 agg_sh.at[dstbig.at[pbp, islotp]],
                                 ssem.at[1 - b], add=True)

    ub = (m + 15) // 16

    r00 = pl.multiple_of(rstart, 8)
    pltpu.async_copy(edges.at[g, 0, pl.ds(r00, 8)], srcbig.at[0], isem)
    pltpu.async_copy(edges.at[g, 1, pl.ds(r00, 8)], dstbig.at[0], isem)

    def body(u, _):
        for q in range(8):
            for b in range(2):
                step(u, q, b)
        return 0
    lax.fori_loop(0, ub, body, 0)

    @pl.when(s < 8)
    def _():
        gwait(1)
        pltpu.async_copy(rows.at[1], agg_sh.at[dstbig.at[1, 7]],
                         ssem.at[1], add=True)

    @pl.when(s >= 8)
    def _():
        gwait(1)
        pltpu.async_copy(rows.at[1], agg_sh.at[dstbig.at[0, 7]],
                         ssem.at[1], add=True)

    @pl.when(s == NS - 1)
    def _():
        for t in range(4):
            b = t % 2
            pltpu.sync_copy(edges.at[g, 0, 2496 + t], srcbig.at[0, t])
            pltpu.sync_copy(edges.at[g, 1, 2496 + t], dstbig.at[0, t])
            adjust(0, t)
            swait(b)
            d = pltpu.async_copy(h2.at[srcbig.at[0, t]], rows.at[b],
                                 gsem.at[b])
            d.wait()
            pltpu.async_copy(rows.at[b], agg_sh.at[dstbig.at[0, t]],
                             ssem.at[b], add=True)

    for b in range(2):
        swait(b)

    plsc.subcore_barrier()

    @pl.when(s < NS - 1)
    def _():
        pltpu.sync_copy(agg_sh.at[pl.ds(start, 624)],
                        out.at[c, pl.ds(start, 624)])

    @pl.when(s == NS - 1)
    def _():
        pltpu.sync_copy(agg_sh.at[pl.ds(9360, 640)],
                        out.at[c, pl.ds(9360, 640)])


def _msg_call(pair, h2, edges):
    mesh = plsc.VectorSubcoreMesh(core_axis_name="c", subcore_axis_name="s")
    return pl.kernel(
        functools.partial(_msg_body, pair),
        out_type=jax.ShapeDtypeStruct((2, N, D), f32),
        mesh=mesh,
        scratch_types=[
            pltpu.VMEM_SHARED((N, D), f32),
            pltpu.VMEM((2, 8, 128), i32),
            pltpu.VMEM((2, 8, 128), i32),
            pltpu.VMEM((2, 128, D), f32),
            pltpu.SemaphoreType.DMA((2,)),
            pltpu.SemaphoreType.DMA((2,)),
            pltpu.SemaphoreType.DMA,
        ],
    )(h2, edges)


def _h_body(col0, col1, x, deg_t, Wa, Wb, out):
    xv = x[...]
    def ns(col):
        d = deg_t[:, col:col + 1]
        return jnp.where(d > 0.0, lax.rsqrt(jnp.maximum(d, 1.0)), 0.0)
    out[0:N, :] = jnp.dot(xv, Wa[...], preferred_element_type=f32) * ns(col0)
    out[N:2 * N, :] = jnp.dot(xv, Wb[...], preferred_element_type=f32) * ns(col1)


def _h_call(col0, col1, x, deg_t, Wa, Wb):
    return pl.pallas_call(
        functools.partial(_h_body, col0, col1),
        out_shape=jax.ShapeDtypeStruct((2 * N, D), f32),
    )(x, deg_t, Wa, Wb)


def _post_body(c0, c1, agg, deg_t, W1, b1, W2, g, b, out):
    def nd(col):
        d = deg_t[:, col:col + 1]
        return jnp.where(d > 0.0, lax.rsqrt(jnp.maximum(d, 1.0)), 0.0)

    def metapath(i, col):
        m = agg[i] * nd(col)
        return jnp.where(m >= 0.0, m, 0.01 * m)

    m0 = metapath(0, c0)
    m1 = metapath(1, c1)

    W1v = W1[...]
    b1v = b1[...]
    W2v = W2[...]

    def score(m):
        t = jnp.tanh(jnp.dot(m, W1v, preferred_element_type=f32) + b1v[None, :])
        w = jnp.dot(t, W2v, preferred_element_type=f32)
        return jnp.mean(w)

    w0 = score(m0)
    w1 = score(m1)
    mx = jnp.maximum(w0, w1)
    e0 = jnp.exp(w0 - mx)
    e1 = jnp.exp(w1 - mx)
    bsum = e0 + e1
    emb = (e0 / bsum) * m0 + (e1 / bsum) * m1
    emb = jnp.maximum(emb, 0.0)
    mu = jnp.mean(emb, axis=-1, keepdims=True)
    var = jnp.mean((emb - mu) ** 2, axis=-1, keepdims=True)
    out[...] = (emb - mu) * lax.rsqrt(var + 1e-5) * g[...][None, :] + b[...][None, :]


def _post_call(c0, c1, agg, deg_t, W1, b1, W2, g, b):
    return pl.pallas_call(
        functools.partial(_post_body, c0, c1),
        out_shape=jax.ShapeDtypeStruct((N, D), f32),
    )(agg, deg_t, W1, b1, W2, g, b)


_BW = 1024 // (NC * NS)


def _gather_body(uemb, iemb, uidx, iidx, nidx, ou, oi, on, idxv, rows, sem):
    c = lax.axis_index("c")
    s = lax.axis_index("s")
    wid = s * NC + c
    base = wid * _BW
    for table, idx_hbm, out in ((uemb, uidx, ou), (iemb, iidx, oi),
                                (iemb, nidx, on)):
        pltpu.sync_copy(idx_hbm.at[pl.ds(base, _BW)], idxv)
        pltpu.async_copy(table.at[idxv], rows, sem).wait()
        pltpu.sync_copy(rows, out.at[pl.ds(base, _BW)])


def _gather_call(uemb, iemb, uidx, iidx, nidx):
    mesh = plsc.VectorSubcoreMesh(core_axis_name="c", subcore_axis_name="s")
    sds = jax.ShapeDtypeStruct((1024, D), f32)
    return pl.kernel(
        _gather_body,
        out_type=(sds, sds, sds),
        mesh=mesh,
        scratch_types=[
            pltpu.VMEM((_BW,), i32),
            pltpu.VMEM((_BW, D), f32),
            pltpu.SemaphoreType.DMA,
        ],
    )(uemb, iemb, uidx, iidx, nidx)


def kernel(user_idx, item_idx, neg_item_idx, up_edges_0, up_edges_1,
           ip_edges_0, ip_edges_1, feat_user, feat_item, W_u0, W_u1, W_i0,
           W_i1, attU_W1, attU_b1, attU_W2, attI_W1, attI_b1, attI_W2,
           ln_g, ln_b):
    edges = jnp.stack([up_edges_0, up_edges_1, ip_edges_0, ip_edges_1])
    edges = edges.astype(i32).reshape(4, 2, ER, 128)

    deg = _deg_call(edges).reshape(8, N)
    deg_t = jnp.transpose(deg)

    h_u = _h_call(0, 2, feat_user, deg_t, W_u0, W_u1)
    h_i = _h_call(4, 6, feat_item, deg_t, W_i0, W_i1)

    agg_u = _msg_call(0, h_u, edges)
    agg_i = _msg_call(1, h_i, edges)

    user_emb = _post_call(1, 3, agg_u, deg_t, attU_W1, attU_b1, attU_W2,
                          ln_g, ln_b)
    item_emb = _post_call(5, 7, agg_i, deg_t, attI_W1, attI_b1, attI_W2,
                          ln_g, ln_b)

    return _gather_call(user_emb, item_emb,
                        user_idx.astype(i32), item_idx.astype(i32),
                        neg_item_idx.astype(i32))
